# baseline (speedup 1.0000x reference)
import functools

import jax
import jax.numpy as jnp
from jax import lax
from jax.experimental import pallas as pl
from jax.experimental.pallas import tpu as pltpu
from jax.experimental.pallas import tpu_sc as plsc

_L = 16
_GRP = 8
_BLK = 128
_PHASE = 128

_N_SC = 8192


def _sc_gather(items_sc, table_t, tail_t, n_full_blocks, tail_base):
    (batch,) = items_sc.shape
    dim = table_t.shape[0]

    idx8 = items_sc.reshape(batch // _GRP, _GRP)
    idx_sp = jnp.pad(idx8, ((0, 0), (0, _L - _GRP))).reshape(-1)

    info = plsc.get_sparse_core_info()
    num_workers = info.num_cores * info.num_subcores
    b_per_w = batch // num_workers
    n_phases = b_per_w // _PHASE
    grp_per_phase = _PHASE // _GRP

    mesh = plsc.VectorSubcoreMesh(core_axis_name="c", subcore_axis_name="s")

    scratch = (
        [pltpu.VMEM((b_per_w * 2,), jnp.int32)]
        + [pltpu.VMEM((dim, _BLK), jnp.float32) for _ in range(2 * _GRP)]
        + [pltpu.VMEM((dim, _BLK), jnp.float32) for _ in range(2)]
        + [pltpu.VMEM((dim, _BLK), jnp.float32)]
        + [pltpu.SemaphoreType.DMA for _ in range(5)]
    )

    @functools.partial(
        pl.kernel,
        mesh=mesh,
        out_type=jax.ShapeDtypeStruct((dim, batch), jnp.float32),
        scratch_types=scratch,
        compiler_params=pltpu.CompilerParams(
            use_tc_tiling_on_sc=True, needs_layout_passes=False
        ),
    )
    def gather_kernel(idx_hbm, table_hbm, tail_hbm, out_hbm, idx_v, *rest):
        slots = rest[: 2 * _GRP]
        cbs = rest[2 * _GRP : 2 * _GRP + 2]
        tail_v = rest[2 * _GRP + 2]
        sem_a, sem_b, sem_cb0, sem_cb1, sem_tail = rest[2 * _GRP + 3 :]
        half_sems = (sem_a, sem_b)
        cb_sems = (sem_cb0, sem_cb1)

        wid = lax.axis_index("s") * info.num_cores + lax.axis_index("c")
        pltpu.sync_copy(idx_hbm.at[pl.ds(wid * b_per_w * 2, b_per_w * 2)], idx_v)
        pltpu.async_copy(tail_hbm, tail_v, sem_tail).wait()

        iota = lax.iota(jnp.int32, _L)

        def lane_info(vec, b):
            v = vec[b]
            c = lax.shift_right_logical(v, 7)
            start = pl.multiple_of(c * _BLK, _BLK)
            return v, start, c < n_full_blocks

        def issue_group(g, half):
            vec = idx_v[pl.ds(g * _L, _L)]
            for b in range(_GRP):
                v, start, main = lane_info(vec, b)

                @pl.when(main)
                def _():
                    pltpu.async_copy(
                        table_hbm.at[:, pl.ds(start, _BLK)],
                        slots[half * _GRP + b],
                        half_sems[half],
                    )

        def drain_extract(g, half, cb):
            vec = idx_v[pl.ds(g * _L, _L)]
            for b in range(_GRP):
                v, start, main = lane_info(vec, b)

                @pl.when(main)
                def _():
                    pltpu.make_async_copy(
                        table_hbm.at[:, pl.ds(start, _BLK)],
                        slots[half * _GRP + b],
                        half_sems[half],
                    ).wait()

            for b in range(_GRP):
                v, start, main = lane_info(vec, b)
                n_loc = (g % grp_per_phase) * _GRP + b
                dst_col = jnp.full((_L,), n_loc, jnp.int32)

                @pl.when(main)
                def _():
                    u = jnp.full((_L,), v & (_BLK - 1), jnp.int32)
                    src = slots[half * _GRP + b]
                    x0 = plsc.load_gather(src, [iota, u])
                    x1 = plsc.load_gather(src, [iota + _L, u])
                    plsc.store_scatter(cb, [iota, dst_col], x0)
                    plsc.store_scatter(cb, [iota + _L, dst_col], x1)

                @pl.when(jnp.logical_not(main))
                def _():
                    ut = jnp.full((_L,), v - tail_base, jnp.int32)
                    x0 = plsc.load_gather(tail_v, [iota, ut])
                    x1 = plsc.load_gather(tail_v, [iota + _L, ut])
                    plsc.store_scatter(cb, [iota, dst_col], x0)
                    plsc.store_scatter(cb, [iota + _L, dst_col], x1)

        def out_win(p):
            col = pl.multiple_of(wid * b_per_w + p * _PHASE, _BLK)
            return out_hbm.at[:, pl.ds(col, _PHASE)]

        for p in range(n_phases):
            cb = cbs[p % 2]
            sem_cb = cb_sems[p % 2]
            if p >= 2:
                pltpu.make_async_copy(cb, out_win(p - 2), sem_cb).wait()
            g0 = p * grp_per_phase
            issue_group(g0, 0)

            def body(k, _):
                ga = g0 + 2 * k
                issue_group(ga + 1, 1)
                drain_extract(ga, 0, cb)

                @pl.when(2 * k + 2 < grp_per_phase)
                def _():
                    issue_group(ga + 2, 0)

                drain_extract(ga + 1, 1, cb)
                return 0

            lax.fori_loop(0, grp_per_phase // 2, body, 0)
            pltpu.async_copy(cb, out_win(p), sem_cb)

        for p in range(max(n_phases - 2, 0), n_phases):
            pltpu.make_async_copy(cbs[p % 2], out_win(p), cb_sems[p % 2]).wait()

    return gather_kernel(idx_sp, table_t, tail_t)


def _tc_gather(items_tc, table_t):
    (n,) = items_tc.shape
    dim = table_t.shape[0]
    n_p = n // _PHASE

    def body(idx_ref, blk_ref, out_ref):
        p = pl.program_id(0)
        j = pl.program_id(1)
        u = idx_ref[p * _PHASE + j] % _BLK
        lane = lax.broadcasted_iota(jnp.int32, (dim, _PHASE), 1)
        x = blk_ref[...]
        col = jnp.sum(jnp.where(lane == u, x, 0.0), axis=1, keepdims=True)

        @pl.when(j == 0)
        def _():
            out_ref[...] = jnp.zeros_like(out_ref)

        out_ref[...] += jnp.where(lane == j, col, 0.0)

    grid_spec = pltpu.PrefetchScalarGridSpec(
        num_scalar_prefetch=1,
        grid=(n_p, _PHASE),
        in_specs=[
            pl.BlockSpec(
                (dim, _BLK), lambda p, j, idx: (0, idx[p * _PHASE + j] // _BLK)
            )
        ],
        out_specs=pl.BlockSpec((dim, _PHASE), lambda p, j, idx: (0, p)),
    )
    return pl.pallas_call(
        body,
        grid_spec=grid_spec,
        out_shape=jax.ShapeDtypeStruct((dim, n), jnp.float32),
        compiler_params=pltpu.CompilerParams(
            dimension_semantics=("arbitrary", "arbitrary")
        ),
    )(items_tc, table_t)


def kernel(items, tf_matrix):
    vocab, dim = tf_matrix.shape
    n_full_blocks = vocab // _BLK
    tail_base = n_full_blocks * _BLK

    table_t = tf_matrix.T
    tail_t = jnp.pad(
        table_t[:, tail_base:], ((0, 0), (0, _BLK - (vocab - tail_base)))
    )

    idx = items.astype(jnp.int32)
    sc_out = _sc_gather(idx[:_N_SC], table_t, tail_t, n_full_blocks, tail_base)
    tc_out = _tc_gather(idx[_N_SC:], table_t)
    return jnp.concatenate([sc_out, tc_out], axis=1).T


# speedup vs baseline: 25.2562x; 25.2562x over previous
import functools

import jax
import jax.numpy as jnp
from jax import lax
from jax.experimental import pallas as pl
from jax.experimental.pallas import tpu as pltpu
from jax.experimental.pallas import tpu_sc as plsc

_L = 16
_GRP = 8
_BLK = 128
_PHASE = 128


def kernel(items, tf_matrix):
    (batch,) = items.shape
    vocab, dim = tf_matrix.shape
    assert dim == 32

    n_full_blocks = vocab // _BLK
    tail_base = n_full_blocks * _BLK

    table_t = tf_matrix.T
    tail_t = jnp.pad(table_t[:, tail_base:], ((0, 0), (0, _BLK - (vocab - tail_base))))

    idx8 = items.astype(jnp.int32).reshape(batch // _GRP, _GRP)
    idx_sp = jnp.pad(idx8, ((0, 0), (0, _L - _GRP))).reshape(-1)

    info = plsc.get_sparse_core_info()
    num_workers = info.num_cores * info.num_subcores
    b_per_w = batch // num_workers
    n_phases = b_per_w // _PHASE
    grp_per_phase = _PHASE // _GRP

    mesh = plsc.VectorSubcoreMesh(core_axis_name="c", subcore_axis_name="s")

    scratch = (
        [pltpu.VMEM((b_per_w * 2,), jnp.int32)]
        + [pltpu.VMEM((dim, _BLK), jnp.float32) for _ in range(2 * _GRP)]
        + [pltpu.VMEM((dim, _BLK), jnp.float32) for _ in range(2)]
        + [pltpu.VMEM((dim, _BLK), jnp.float32)]
        + [pltpu.SemaphoreType.DMA for _ in range(5)]
    )

    @functools.partial(
        pl.kernel,
        mesh=mesh,
        out_type=jax.ShapeDtypeStruct((dim, batch), jnp.float32),
        scratch_types=scratch,
        compiler_params=pltpu.CompilerParams(
            use_tc_tiling_on_sc=True, needs_layout_passes=False
        ),
    )
    def gather_kernel(idx_hbm, table_hbm, tail_hbm, out_hbm, idx_v, *rest):
        slots = rest[: 2 * _GRP]
        cbs = rest[2 * _GRP : 2 * _GRP + 2]
        tail_v = rest[2 * _GRP + 2]
        sem_a, sem_b, sem_cb0, sem_cb1, sem_tail = rest[2 * _GRP + 3 :]
        half_sems = (sem_a, sem_b)
        cb_sems = (sem_cb0, sem_cb1)

        wid = lax.axis_index("s") * info.num_cores + lax.axis_index("c")
        pltpu.sync_copy(idx_hbm.at[pl.ds(wid * b_per_w * 2, b_per_w * 2)], idx_v)
        pltpu.async_copy(tail_hbm, tail_v, sem_tail).wait()

        iota = lax.iota(jnp.int32, _L)

        def lane_info(vec, b):
            v = vec[b]
            c = lax.shift_right_logical(v, 7)
            start = pl.multiple_of(c * _BLK, _BLK)
            return v, start, c < n_full_blocks

        def issue_group(g, half):
            vec = idx_v[pl.ds(g * _L, _L)]
            for b in range(_GRP):
                v, start, main = lane_info(vec, b)

                @pl.when(main)
                def _():
                    pltpu.async_copy(
                        table_hbm.at[:, pl.ds(start, _BLK)],
                        slots[half * _GRP + b],
                        half_sems[half],
                    )

        def drain_extract(g, half, cb):
            vec = idx_v[pl.ds(g * _L, _L)]
            for b in range(_GRP):
                v, start, main = lane_info(vec, b)

                @pl.when(main)
                def _():
                    pltpu.make_async_copy(
                        table_hbm.at[:, pl.ds(start, _BLK)],
                        slots[half * _GRP + b],
                        half_sems[half],
                    ).wait()

            for b in range(_GRP):
                v, start, main = lane_info(vec, b)
                n_loc = (g % grp_per_phase) * _GRP + b
                dst_col = jnp.full((_L,), n_loc, jnp.int32)

                @pl.when(main)
                def _():
                    u = jnp.full((_L,), v & (_BLK - 1), jnp.int32)
                    src = slots[half * _GRP + b]
                    x0 = plsc.load_gather(src, [iota, u])
                    x1 = plsc.load_gather(src, [iota + _L, u])
                    plsc.store_scatter(cb, [iota, dst_col], x0)
                    plsc.store_scatter(cb, [iota + _L, dst_col], x1)

                @pl.when(jnp.logical_not(main))
                def _():
                    ut = jnp.full((_L,), v - tail_base, jnp.int32)
                    x0 = plsc.load_gather(tail_v, [iota, ut])
                    x1 = plsc.load_gather(tail_v, [iota + _L, ut])
                    plsc.store_scatter(cb, [iota, dst_col], x0)
                    plsc.store_scatter(cb, [iota + _L, dst_col], x1)

        def out_win(p):
            col = pl.multiple_of(wid * b_per_w + p * _PHASE, _BLK)
            return out_hbm.at[:, pl.ds(col, _PHASE)]

        for p in range(n_phases):
            cb = cbs[p % 2]
            sem_cb = cb_sems[p % 2]
            if p >= 2:
                pltpu.make_async_copy(cb, out_win(p - 2), sem_cb).wait()
            g0 = p * grp_per_phase
            issue_group(g0, 0)

            def body(k, _):
                ga = g0 + 2 * k
                issue_group(ga + 1, 1)
                drain_extract(ga, 0, cb)

                @pl.when(2 * k + 2 < grp_per_phase)
                def _():
                    issue_group(ga + 2, 0)

                drain_extract(ga + 1, 1, cb)
                return 0

            lax.fori_loop(0, grp_per_phase // 2, body, 0)
            pltpu.async_copy(cb, out_win(p), sem_cb)

        pltpu.make_async_copy(cbs[0], out_win(n_phases - 2), sem_cb0).wait()
        pltpu.make_async_copy(cbs[1], out_win(n_phases - 1), sem_cb1).wait()

    return gather_kernel(idx_sp, table_t, tail_t).T
